# Initial kernel scaffold; baseline (speedup 1.0000x reference)
import jax
import jax.numpy as jnp
from jax import lax
from jax.experimental import pallas as pl
from jax.experimental.pallas import tpu as pltpu
from jax.experimental.pallas import tpu_sc as plsc

N = 10000
D = 256
E = 160000
NC = 2
NS = 16
LANES = 16
DH = D // NC
NPAD = 10240
RPT = NPAD // NS
EPT = E // NS
ECHUNK = 128
NFULL = EPT // ECHUNK
NCHUNK = 80
ACC_ROWS = 10496
ZPT = ACC_ROWS // NS
TRASH = NPAD


def _rsqrt16(x):
    i = plsc.bitcast(x, jnp.int32)
    i = jnp.int32(0x5F3759DF) - lax.shift_right_logical(i, 1)
    y = plsc.bitcast(i, jnp.float32)
    for _ in range(3):
        y = y * (1.5 - 0.5 * x * y * y)
    return y


def _sc_body(emb_sr, emb_tg, ssr, dsr, stg, dtg, v2, ubh,
             acc, hists_sp, sidx, didx, hist, hbuf, dbuf, rbuf,
             gbuf, zbuf, abuf, uvbuf, obuf, xbuf, gsem):
    c = lax.axis_index("c")
    t = lax.axis_index("s")
    ebase = t * EPT
    rbase = t * RPT
    nblk = jnp.where(t == NS - 1, (N - (NS - 1) * RPT) // LANES, RPT // LANES)

    zero16 = jnp.zeros((LANES,), jnp.float32)
    one16 = jnp.full((LANES,), 1.0, jnp.float32)
    iota16 = lax.iota(jnp.int32, LANES)

    def _zb(i, _):
        zbuf[i // 8, pl.ds((i % 8) * LANES, LANES)] = zero16
        return 0
    lax.fori_loop(0, 16 * 8, _zb, 0)

    for g in range(2):
        emb = (emb_sr, emb_tg)[g]
        sref = (ssr, stg)[g]
        dref = (dsr, dtg)[g]

        for idxv, ref in ((didx, dref), (sidx, sref)):
            def _cp(j, _, idxv=idxv, ref=ref):
                pltpu.sync_copy(ref.at[pl.ds(ebase + j * ECHUNK, ECHUNK)],
                                idxv.at[j])
                return 0
            lax.fori_loop(0, NFULL, _cp, 0)
            pltpu.sync_copy(ref.at[pl.ds(ebase + NFULL * ECHUNK, 16)],
                            idxv.at[NFULL, pl.ds(0, 16)])
        for k in range(1, 8):
            didx[NFULL, pl.ds(k * 16, 16)] = TRASH + (k - 1) * 16 + iota16
            sidx[NFULL, pl.ds(k * 16, 16)] = N + (k - 1) * 16 + iota16
        for k in range(8):
            didx[NFULL + 1, pl.ds(k * 16, 16)] = TRASH + 112 + k * 16 + iota16
            sidx[NFULL + 1, pl.ds(k * 16, 16)] = N + 112 + k * 16 + iota16

        def _zh(i, _):
            hist[pl.ds(i * LANES, LANES)] = zero16
            return 0
        lax.fori_loop(0, NPAD // LANES, _zh, 0)

        def _hs(i, _):
            v = didx[i // 8, pl.ds((i % 8) * LANES, LANES)]
            plsc.addupdate_scatter(hist, [v], one16)
            return 0
        lax.fori_loop(0, EPT // LANES, _hs, 0)

        plsc.subcore_barrier()
        pltpu.sync_copy(hist, hists_sp.at[t])
        plsc.subcore_barrier()
        pltpu.sync_copy(hists_sp.at[:, pl.ds(rbase, RPT)], hbuf)

        def _sum(kk, _):
            s = one16
            for tt in range(NS):
                s = s + hbuf[tt, pl.ds(kk * LANES, LANES)]
            dbuf[pl.ds(kk * LANES, LANES)] = s
            return 0
        lax.fori_loop(0, RPT // LANES, _sum, 0)

        def _rs(kk, _):
            rbuf[pl.ds(kk * LANES, LANES)] = _rsqrt16(
                dbuf[pl.ds(kk * LANES, LANES)])
            return 0
        lax.fori_loop(0, RPT // LANES, _rs, 0)

        def _u0(b, _):
            r0 = rbase + b * LANES
            pltpu.sync_copy(emb.at[pl.ds(r0, LANES), pl.ds(c * DH, DH)], xbuf)

            def _row(i, _):
                s = plsc.load_gather(
                    rbuf, [jnp.full((LANES,), b * LANES + i, jnp.int32)])
                for k in range(DH // LANES):
                    obuf[i, pl.ds(k * LANES, LANES)] = (
                        xbuf[i, pl.ds(k * LANES, LANES)] * s)
                return 0
            lax.fori_loop(0, LANES, _row, 0)
            pltpu.sync_copy(obuf, ubh.at[0].at[c].at[pl.ds(r0, LANES)])
            return 0
        lax.fori_loop(0, nblk, _u0, 0)

        for l in range(2):
            plsc.subcore_barrier()

            def _za(k2, _):
                pltpu.sync_copy(zbuf, acc.at[pl.ds(t * ZPT + k2 * 16, 16)])
                return 0
            lax.fori_loop(0, ZPT // 16, _za, 0)
            plsc.subcore_barrier()

            uin = ubh.at[l].at[c]

            def _gs(j, _, uin=uin):
                pltpu.async_copy(uin.at[sidx.at[j]], gbuf, gsem).wait()
                pltpu.sync_copy(gbuf, acc.at[didx.at[j]], add=True)
                return 0
            lax.fori_loop(0, NCHUNK, _gs, 0)
            plsc.subcore_barrier()

            def _ep(b, _, uin=uin, l=l, g=g):
                r0 = rbase + b * LANES
                pltpu.sync_copy(acc.at[pl.ds(r0, LANES)], abuf)
                pltpu.sync_copy(uin.at[pl.ds(r0, LANES)], uvbuf)

                def _row(i, _):
                    if l == 0:
                        dd = plsc.load_gather(
                            dbuf,
                            [jnp.full((LANES,), b * LANES + i, jnp.int32)])
                        s = 1.0 / dd
                        for k in range(DH // LANES):
                            obuf[i, pl.ds(k * LANES, LANES)] = (
                                abuf[i, pl.ds(k * LANES, LANES)]
                                + uvbuf[i, pl.ds(k * LANES, LANES)]) * s
                    else:
                        for k in range(DH // LANES):
                            obuf[i, pl.ds(k * LANES, LANES)] = (
                                abuf[i, pl.ds(k * LANES, LANES)]
                                + uvbuf[i, pl.ds(k * LANES, LANES)])
                    return 0
                lax.fori_loop(0, LANES, _row, 0)
                if l == 0:
                    pltpu.sync_copy(obuf, ubh.at[1].at[c].at[pl.ds(r0, LANES)])
                else:
                    pltpu.sync_copy(obuf, v2.at[g].at[c].at[pl.ds(r0, LANES)])
                return 0
            lax.fori_loop(0, nblk, _ep, 0)


_sc_gcn = pl.kernel(
    _sc_body,
    out_type=(
        jax.ShapeDtypeStruct((2, NC, NPAD, DH), jnp.float32),
        jax.ShapeDtypeStruct((2, NC, NPAD, DH), jnp.float32),
    ),
    mesh=plsc.VectorSubcoreMesh(core_axis_name="c", subcore_axis_name="s",
                                num_cores=NC, num_subcores=NS),
    scratch_types=[
        pltpu.VMEM_SHARED((ACC_ROWS, DH), jnp.float32),
        pltpu.VMEM_SHARED((NS, NPAD), jnp.float32),
        pltpu.VMEM((NCHUNK, ECHUNK), jnp.int32),
        pltpu.VMEM((NCHUNK, ECHUNK), jnp.int32),
        pltpu.VMEM((NPAD,), jnp.float32),
        pltpu.VMEM((NS, RPT), jnp.float32),
        pltpu.VMEM((RPT,), jnp.float32),
        pltpu.VMEM((RPT,), jnp.float32),
        pltpu.VMEM((ECHUNK, DH), jnp.float32),
        pltpu.VMEM((16, DH), jnp.float32),
        pltpu.VMEM((16, DH), jnp.float32),
        pltpu.VMEM((16, DH), jnp.float32),
        pltpu.VMEM((16, DH), jnp.float32),
        pltpu.VMEM((16, DH), jnp.float32),
        pltpu.SemaphoreType.DMA,
    ],
)


BR = 1000


def _norm_kernel(v_ref, sr_ref, tg_ref):
    v = v_ref[...]
    for gi, oref in ((0, sr_ref), (1, tg_ref)):
        x = jnp.concatenate([v[gi, 0], v[gi, 1]], axis=1)
        nrm = jnp.sqrt(jnp.sum(x * x, axis=1, keepdims=True))
        oref[...] = x / jnp.maximum(nrm, 1e-12)


_norm = pl.pallas_call(
    _norm_kernel,
    grid=(N // BR,),
    in_specs=[pl.BlockSpec((2, NC, BR, DH), lambda i: (0, 0, i, 0))],
    out_specs=[pl.BlockSpec((BR, D), lambda i: (i, 0)),
               pl.BlockSpec((BR, D), lambda i: (i, 0))],
    out_shape=[jax.ShapeDtypeStruct((N, D), jnp.float32)] * 2,
)


def kernel(emb_sr, emb_tg, edge_index_sr, edge_index_tg):
    ssr = edge_index_sr[0].astype(jnp.int32)
    dsr = edge_index_sr[1].astype(jnp.int32)
    stg = edge_index_tg[0].astype(jnp.int32)
    dtg = edge_index_tg[1].astype(jnp.int32)
    v2, _ = _sc_gcn(emb_sr, emb_tg, ssr, dsr, stg, dtg)
    sr, tg = _norm(v2)
    return (sr, tg)


# speedup vs baseline: 7.9442x; 7.9442x over previous
import jax
import jax.numpy as jnp
from jax import lax
from jax.experimental import pallas as pl
from jax.experimental.pallas import tpu as pltpu
from jax.experimental.pallas import tpu_sc as plsc

N = 10000
D = 256
E = 160000
NC = 2
NS = 16
LANES = 16
DH = D // NC
NPAD = 10240
RPT = NPAD // NS
EPT = E // NS
ECHUNK = 128
NFULL = EPT // ECHUNK
NCHUNK = 80
ACC_ROWS = 10496
ZPT = ACC_ROWS // NS
TRASH = NPAD
HR = NPAD // 128


def _rsqrt16(x):
    i = plsc.bitcast(x, jnp.int32)
    i = jnp.int32(0x5F3759DF) - lax.shift_right_logical(i, 1)
    y = plsc.bitcast(i, jnp.float32)
    for _ in range(3):
        y = y * (1.5 - 0.5 * x * y * y)
    return y


def _deg_body(dsr, dtg, degout, hists_sp, hist, mbuf, dbuf):
    c = lax.axis_index("c")
    t = lax.axis_index("s")
    ebase = t * EPT
    rbase = t * RPT
    zero16 = jnp.zeros((LANES,), jnp.float32)
    one16 = jnp.full((LANES,), 1.0, jnp.float32)

    for g in range(2):
        dref = (dsr, dtg)[g]

        def _zh(i, _):
            hist[i // 8, pl.ds((i % 8) * LANES, LANES)] = zero16
            return 0
        lax.fori_loop(0, NPAD // LANES, _zh, 0)

        def _hs(j, _, dref=dref):
            pltpu.sync_copy(dref.at[pl.ds(ebase + j * 128, 128)], mbuf.at[0])

            def _hv(k, _):
                v = mbuf[0, pl.ds(k * LANES, LANES)]
                r = lax.shift_right_logical(v, 7)
                cc = lax.bitwise_and(v, 127)
                plsc.addupdate_scatter(hist, [r, cc], one16)
                return 0
            lax.fori_loop(0, 8, _hv, 0)
            return 0
        lax.fori_loop(0, EPT // 128, _hs, 0)
        pltpu.sync_copy(dref.at[pl.ds(ebase + (EPT // 128) * 128, 16)],
                        mbuf.at[0, pl.ds(0, 16)])
        v = mbuf[0, pl.ds(0, 16)]
        plsc.addupdate_scatter(hist, [lax.shift_right_logical(v, 7),
                                      lax.bitwise_and(v, 127)], one16)

        plsc.subcore_barrier()
        pltpu.sync_copy(hist, hists_sp.at[pl.ds(t * HR, HR)])
        plsc.subcore_barrier()

        def _sum(kk, _):
            s = one16
            for tt in range(NS):
                s = s + hist[tt * 5 + kk // 8, pl.ds((kk % 8) * LANES, LANES)]
            dbuf[pl.ds(kk * LANES, LANES)] = s
            return 0
        for tt in range(NS):
            pltpu.sync_copy(hists_sp.at[pl.ds(tt * HR + t * 5, 5)],
                            hist.at[pl.ds(tt * 5, 5)])
        lax.fori_loop(0, RPT // LANES, _sum, 0)
        @pl.when(c == 0)
        def _():
            pltpu.sync_copy(dbuf, degout.at[g].at[pl.ds(rbase, RPT)])
        plsc.subcore_barrier()


_deg = pl.kernel(
    _deg_body,
    out_type=jax.ShapeDtypeStruct((2, NPAD), jnp.float32),
    mesh=plsc.VectorSubcoreMesh(core_axis_name="c", subcore_axis_name="s",
                                num_cores=NC, num_subcores=NS),
    compiler_params=pltpu.CompilerParams(needs_layout_passes=False),
    scratch_types=[
        pltpu.VMEM_SHARED((NS * HR, 128), jnp.float32),
        pltpu.VMEM((HR, 128), jnp.float32),
        pltpu.VMEM((1, 128), jnp.int32),
        pltpu.VMEM((RPT,), jnp.float32),
    ],
)


def _sc_body(emb_sr, emb_tg, ssr, dsr, stg, dtg, deg, v2, ubh,
             acc, sidx, didx, dbuf, rbuf, gbuf, gsem):
    c = lax.axis_index("c")
    t = lax.axis_index("s")
    ebase = t * EPT
    rbase = t * RPT
    nblk = jnp.where(t == NS - 1, (N - (NS - 1) * RPT) // LANES, RPT // LANES)

    zero16 = jnp.zeros((LANES,), jnp.float32)
    one16 = jnp.full((LANES,), 1.0, jnp.float32)
    iota16 = lax.iota(jnp.int32, LANES)

    ZB, AB, UB, OB = 0, 16, 32, 48

    for g in range(2):
        emb = (emb_sr, emb_tg)[g]
        sref = (ssr, stg)[g]
        dref = (dsr, dtg)[g]

        for idxv, ref in ((didx, dref), (sidx, sref)):
            def _cp(j, _, idxv=idxv, ref=ref):
                pltpu.sync_copy(ref.at[pl.ds(ebase + j * ECHUNK, ECHUNK)],
                                idxv.at[j])
                return 0
            lax.fori_loop(0, NFULL, _cp, 0)
            pltpu.sync_copy(ref.at[pl.ds(ebase + NFULL * ECHUNK, 16)],
                            idxv.at[NFULL, pl.ds(0, 16)])
        pc = 0
        for r in range(NFULL, NCHUNK):
            for k in range(ECHUNK // 16):
                if r == NFULL and k == 0:
                    continue
                didx[r, pl.ds(k * 16, 16)] = TRASH + pc * 16 + iota16
                sidx[r, pl.ds(k * 16, 16)] = N + pc * 16 + iota16
                pc += 1

        pltpu.sync_copy(deg.at[g].at[pl.ds(rbase, RPT)], dbuf)

        def _rs(kk, _):
            rbuf[pl.ds(kk * LANES, LANES)] = _rsqrt16(
                dbuf[pl.ds(kk * LANES, LANES)])
            return 0
        lax.fori_loop(0, RPT // LANES, _rs, 0)

        def _u0(b, _):
            r0 = rbase + b * LANES
            pltpu.sync_copy(emb.at[pl.ds(r0, LANES), pl.ds(c * DH, DH)],
                            gbuf.at[pl.ds(ZB, 16)])

            def _row(i, _):
                s = plsc.load_gather(
                    rbuf, [jnp.full((LANES,), b * LANES + i, jnp.int32)])
                for k in range(DH // LANES):
                    gbuf[OB + i, pl.ds(k * LANES, LANES)] = (
                        gbuf[ZB + i, pl.ds(k * LANES, LANES)] * s)
                return 0
            lax.fori_loop(0, LANES, _row, 0)
            pltpu.sync_copy(gbuf.at[pl.ds(OB, 16)],
                            ubh.at[0].at[c].at[pl.ds(r0, LANES)])
            return 0
        lax.fori_loop(0, nblk, _u0, 0)

        for l in range(2):
            plsc.subcore_barrier()

            def _zb(i, _):
                gbuf[ZB + i // 8, pl.ds((i % 8) * LANES, LANES)] = zero16
                return 0
            lax.fori_loop(0, 16 * 8, _zb, 0)

            def _za(k2, _):
                pltpu.sync_copy(gbuf.at[pl.ds(ZB, 16)],
                                acc.at[pl.ds(t * ZPT + k2 * 16, 16)])
                return 0
            lax.fori_loop(0, ZPT // 16, _za, 0)
            plsc.subcore_barrier()

            uin = ubh.at[l].at[c]

            def _gs(j, _, uin=uin):
                pltpu.async_copy(uin.at[sidx.at[j]],
                                 gbuf.at[pl.ds(0, ECHUNK)], gsem).wait()
                pltpu.sync_copy(gbuf.at[pl.ds(0, ECHUNK)],
                                acc.at[didx.at[j]], add=True)
                return 0
            lax.fori_loop(0, NCHUNK, _gs, 0)
            plsc.subcore_barrier()

            def _ep(b, _, uin=uin, l=l, g=g):
                r0 = rbase + b * LANES
                pltpu.sync_copy(acc.at[pl.ds(r0, LANES)],
                                gbuf.at[pl.ds(AB, 16)])
                pltpu.sync_copy(uin.at[pl.ds(r0, LANES)],
                                gbuf.at[pl.ds(UB, 16)])

                def _row(i, _):
                    if l == 0:
                        dd = plsc.load_gather(
                            dbuf,
                            [jnp.full((LANES,), b * LANES + i, jnp.int32)])
                        s = 1.0 / dd
                        for k in range(DH // LANES):
                            gbuf[OB + i, pl.ds(k * LANES, LANES)] = (
                                gbuf[AB + i, pl.ds(k * LANES, LANES)]
                                + gbuf[UB + i, pl.ds(k * LANES, LANES)]) * s
                    else:
                        for k in range(DH // LANES):
                            gbuf[OB + i, pl.ds(k * LANES, LANES)] = (
                                gbuf[AB + i, pl.ds(k * LANES, LANES)]
                                + gbuf[UB + i, pl.ds(k * LANES, LANES)])
                    return 0
                lax.fori_loop(0, LANES, _row, 0)
                if l == 0:
                    pltpu.sync_copy(gbuf.at[pl.ds(OB, 16)],
                                    ubh.at[1].at[c].at[pl.ds(r0, LANES)])
                else:
                    pltpu.sync_copy(gbuf.at[pl.ds(OB, 16)],
                                    v2.at[g].at[c].at[pl.ds(r0, LANES)])
                return 0
            lax.fori_loop(0, nblk, _ep, 0)


_sc_gcn = pl.kernel(
    _sc_body,
    out_type=(
        jax.ShapeDtypeStruct((2, NC, NPAD, DH), jnp.float32),
        jax.ShapeDtypeStruct((2, NC, NPAD, DH), jnp.float32),
    ),
    mesh=plsc.VectorSubcoreMesh(core_axis_name="c", subcore_axis_name="s",
                                num_cores=NC, num_subcores=NS),
    compiler_params=pltpu.CompilerParams(needs_layout_passes=False),
    scratch_types=[
        pltpu.VMEM_SHARED((ACC_ROWS, DH), jnp.float32),
        pltpu.VMEM((NCHUNK, ECHUNK), jnp.int32),
        pltpu.VMEM((NCHUNK, ECHUNK), jnp.int32),
        pltpu.VMEM((RPT,), jnp.float32),
        pltpu.VMEM((RPT,), jnp.float32),
        pltpu.VMEM((ECHUNK, DH), jnp.float32),
        pltpu.SemaphoreType.DMA,
    ],
)


BR = 1000


def _norm_kernel(v_ref, sr_ref, tg_ref):
    v = v_ref[...]
    for gi, oref in ((0, sr_ref), (1, tg_ref)):
        x = jnp.concatenate([v[gi, 0], v[gi, 1]], axis=1)
        nrm = jnp.sqrt(jnp.sum(x * x, axis=1, keepdims=True))
        oref[...] = x / jnp.maximum(nrm, 1e-12)


_norm = pl.pallas_call(
    _norm_kernel,
    grid=(N // BR,),
    in_specs=[pl.BlockSpec((2, NC, BR, DH), lambda i: (0, 0, i, 0))],
    out_specs=[pl.BlockSpec((BR, D), lambda i: (i, 0)),
               pl.BlockSpec((BR, D), lambda i: (i, 0))],
    out_shape=[jax.ShapeDtypeStruct((N, D), jnp.float32)] * 2,
)


def kernel(emb_sr, emb_tg, edge_index_sr, edge_index_tg):
    ssr = edge_index_sr[0].astype(jnp.int32)
    dsr = edge_index_sr[1].astype(jnp.int32)
    stg = edge_index_tg[0].astype(jnp.int32)
    dtg = edge_index_tg[1].astype(jnp.int32)
    deg = _deg(dsr, dtg)
    v2, _ = _sc_gcn(emb_sr, emb_tg, ssr, dsr, stg, dtg, deg)
    sr, tg = _norm(v2)
    return (sr, tg)


# speedup vs baseline: 11.6159x; 1.4622x over previous
import jax
import jax.numpy as jnp
from jax import lax
from jax.experimental import pallas as pl
from jax.experimental.pallas import tpu as pltpu
from jax.experimental.pallas import tpu_sc as plsc

N = 10000
D = 256
E = 160000
NC = 2
NS = 16
LANES = 16
DH = D // NC
NPAD = 10240
RPT = NPAD // NS
EPT = E // NS
ECHUNK = 128
NCHUNK = 80
EPAD = NCHUNK * ECHUNK - EPT
ACC_ROWS = 10496
ZPT = ACC_ROWS // NS
TRASH = NPAD
HR = NPAD // 128
HRP = 88


def _rsqrt16(x):
    i = plsc.bitcast(x, jnp.int32)
    i = jnp.int32(0x5F3759DF) - lax.shift_right_logical(i, 1)
    y = plsc.bitcast(i, jnp.float32)
    for _ in range(3):
        y = y * (1.5 - 0.5 * x * y * y)
    return y


def _deg_body(dp0, dp1, degout, hists_sp, hist, idxbuf, dbuf):
    c = lax.axis_index("c")
    t = lax.axis_index("s")
    rbase = t * RPT
    zero16 = jnp.zeros((LANES,), jnp.float32)
    one16 = jnp.full((LANES,), 1.0, jnp.float32)

    for g in range(2):
        dp = (dp0, dp1)[g]

        def _zh(i, _):
            hist[i // 8, pl.ds((i % 8) * LANES, LANES)] = zero16
            return 0
        lax.fori_loop(0, HRP * 8, _zh, 0)

        pltpu.sync_copy(dp.at[pl.ds(t * NCHUNK, NCHUNK)], idxbuf)

        def _hs(i, _):
            v = idxbuf[i // 8, pl.ds((i % 8) * LANES, LANES)]
            r = lax.shift_right_logical(v, 7)
            cc = lax.bitwise_and(v, 127)
            plsc.addupdate_scatter(hist, [r, cc], one16)
            return 0
        lax.fori_loop(0, NCHUNK * 8, _hs, 0)

        plsc.subcore_barrier()
        pltpu.sync_copy(hist.at[pl.ds(0, HR)], hists_sp.at[pl.ds(t * HR, HR)])
        plsc.subcore_barrier()
        for tt in range(NS):
            pltpu.sync_copy(hists_sp.at[pl.ds(tt * HR + t * 5, 5)],
                            hist.at[pl.ds(tt * 5, 5)])

        def _sum(kk, _):
            s = one16
            for tt in range(NS):
                s = s + hist[tt * 5 + kk // 8, pl.ds((kk % 8) * LANES, LANES)]
            dbuf[pl.ds(kk * LANES, LANES)] = s
            return 0
        lax.fori_loop(0, RPT // LANES, _sum, 0)
        @pl.when(c == 0)
        def _():
            pltpu.sync_copy(dbuf, degout.at[g].at[pl.ds(rbase, RPT)])
        plsc.subcore_barrier()


_deg = pl.kernel(
    _deg_body,
    out_type=jax.ShapeDtypeStruct((2, NPAD), jnp.float32),
    mesh=plsc.VectorSubcoreMesh(core_axis_name="c", subcore_axis_name="s",
                                num_cores=NC, num_subcores=NS),
    compiler_params=pltpu.CompilerParams(needs_layout_passes=False),
    scratch_types=[
        pltpu.VMEM_SHARED((NS * HR, 128), jnp.float32),
        pltpu.VMEM((HRP, 128), jnp.float32),
        pltpu.VMEM((NCHUNK, ECHUNK), jnp.int32),
        pltpu.VMEM((RPT,), jnp.float32),
    ],
)


def _sc_body(emb_sr, emb_tg, sp0, dp0, sp1, dp1, deg, v2, ubh,
             acc, dgbuf, rbuf, ga, gb, gi, si,
             is0, is1, is2, is3, gsem, ss0, ss1):
    c = lax.axis_index("c")
    t = lax.axis_index("s")
    rbase = t * RPT
    cbase = t * NCHUNK
    nblk = jnp.where(t == NS - 1, (N - (NS - 1) * RPT) // LANES, RPT // LANES)

    zero16 = jnp.zeros((LANES,), jnp.float32)
    isems = (is0, is1, is2, is3)
    gbufs = (ga, gb)
    ssems = (ss0, ss1)
    ZB, AB, UB, OB = 0, 16, 32, 48

    for g in range(2):
        emb = (emb_sr, emb_tg)[g]
        sp = (sp0, sp1)[g]
        dp = (dp0, dp1)[g]

        pltpu.sync_copy(deg.at[g].at[pl.ds(rbase, RPT)], dgbuf)

        def _rs(kk, _):
            rbuf[pl.ds(kk * LANES, LANES)] = _rsqrt16(
                dgbuf[pl.ds(kk * LANES, LANES)])
            return 0
        lax.fori_loop(0, RPT // LANES, _rs, 0)

        def _u0(b, _):
            r0 = rbase + b * LANES
            pltpu.sync_copy(emb.at[pl.ds(r0, LANES), pl.ds(c * DH, DH)],
                            ga.at[pl.ds(ZB, 16)])

            def _row(i, _):
                s = plsc.load_gather(
                    rbuf, [jnp.full((LANES,), b * LANES + i, jnp.int32)])
                for k in range(DH // LANES):
                    ga[OB + i, pl.ds(k * LANES, LANES)] = (
                        ga[ZB + i, pl.ds(k * LANES, LANES)] * s)
                return 0
            lax.fori_loop(0, LANES, _row, 0)
            pltpu.sync_copy(ga.at[pl.ds(OB, 16)],
                            ubh.at[0].at[c].at[pl.ds(r0, LANES)])
            return 0
        lax.fori_loop(0, nblk, _u0, 0)

        for l in range(2):
            plsc.subcore_barrier()

            def _zb(i, _):
                ga[ZB + i // 8, pl.ds((i % 8) * LANES, LANES)] = zero16
                return 0
            lax.fori_loop(0, 16 * 8, _zb, 0)

            def _za(k2, _):
                pltpu.sync_copy(ga.at[pl.ds(ZB, 16)],
                                acc.at[pl.ds(t * ZPT + k2 * 16, 16)])
                return 0
            lax.fori_loop(0, ZPT // 16, _za, 0)
            plsc.subcore_barrier()

            uin = ubh.at[l].at[c]
            hbm_dummy = uin.at[pl.ds(0, ECHUNK)]

            def issue_idx(j, q, sp=sp, dp=dp):
                pltpu.async_copy(sp.at[cbase + j], gi.at[q], isems[q])
                pltpu.async_copy(dp.at[cbase + j], si.at[q], isems[q])

            def do_chunk(j, slot, prefetch, wait_scatter,
                         uin=uin, hbm_dummy=hbm_dummy, sp=sp, dp=dp,
                         issue_idx=issue_idx):
                p = slot % 2
                q = slot % 4
                if wait_scatter:
                    pltpu.make_async_copy(hbm_dummy, gbufs[p],
                                          ssems[p]).wait()
                pltpu.make_async_copy(sp.at[0], gi.at[q], isems[q]).wait()
                pltpu.make_async_copy(dp.at[0], si.at[q], isems[q]).wait()
                gd = pltpu.async_copy(uin.at[gi.at[q]], gbufs[p], gsem)
                if prefetch is not None:
                    issue_idx(prefetch, (q + 2) % 4)
                gd.wait()
                pltpu.async_copy(gbufs[p], acc.at[si.at[q]], ssems[p],
                                 add=True)

            issue_idx(0, 0)
            issue_idx(1, 1)
            do_chunk(0, 0, 2, False)
            do_chunk(1, 1, 3, False)

            def _pipe(jo, _):
                jb = 2 + jo * 4
                for b in range(4):
                    do_chunk(jb + b, 2 + b, jb + b + 2, True)
                return 0
            lax.fori_loop(0, (NCHUNK - 4) // 4, _pipe, 0)

            do_chunk(NCHUNK - 2, 2, None, True)
            do_chunk(NCHUNK - 1, 3, None, True)
            pltpu.make_async_copy(hbm_dummy, ga, ss0).wait()
            pltpu.make_async_copy(hbm_dummy, gb, ss1).wait()
            plsc.subcore_barrier()

            def _ep(b, _, uin=uin, l=l, g=g):
                r0 = rbase + b * LANES
                pltpu.sync_copy(acc.at[pl.ds(r0, LANES)],
                                ga.at[pl.ds(AB, 16)])
                pltpu.sync_copy(uin.at[pl.ds(r0, LANES)],
                                ga.at[pl.ds(UB, 16)])

                def _row(i, _):
                    if l == 0:
                        dd = plsc.load_gather(
                            dgbuf,
                            [jnp.full((LANES,), b * LANES + i, jnp.int32)])
                        s = 1.0 / dd
                        for k in range(DH // LANES):
                            ga[OB + i, pl.ds(k * LANES, LANES)] = (
                                ga[AB + i, pl.ds(k * LANES, LANES)]
                                + ga[UB + i, pl.ds(k * LANES, LANES)]) * s
                    else:
                        for k in range(DH // LANES):
                            ga[OB + i, pl.ds(k * LANES, LANES)] = (
                                ga[AB + i, pl.ds(k * LANES, LANES)]
                                + ga[UB + i, pl.ds(k * LANES, LANES)])
                    return 0
                lax.fori_loop(0, LANES, _row, 0)
                if l == 0:
                    pltpu.sync_copy(ga.at[pl.ds(OB, 16)],
                                    ubh.at[1].at[c].at[pl.ds(r0, LANES)])
                else:
                    pltpu.sync_copy(ga.at[pl.ds(OB, 16)],
                                    v2.at[g].at[c].at[pl.ds(r0, LANES)])
                return 0
            lax.fori_loop(0, nblk, _ep, 0)


_sc_gcn = pl.kernel(
    _sc_body,
    out_type=(
        jax.ShapeDtypeStruct((2, NC, NPAD, DH), jnp.float32),
        jax.ShapeDtypeStruct((2, NC, NPAD, DH), jnp.float32),
    ),
    mesh=plsc.VectorSubcoreMesh(core_axis_name="c", subcore_axis_name="s",
                                num_cores=NC, num_subcores=NS),
    compiler_params=pltpu.CompilerParams(needs_layout_passes=False),
    scratch_types=[
        pltpu.VMEM_SHARED((ACC_ROWS, DH), jnp.float32),
        pltpu.VMEM((RPT,), jnp.float32),
        pltpu.VMEM((RPT,), jnp.float32),
        pltpu.VMEM((ECHUNK, DH), jnp.float32),
        pltpu.VMEM((ECHUNK, DH), jnp.float32),
        pltpu.VMEM((4, ECHUNK), jnp.int32),
        pltpu.VMEM((4, ECHUNK), jnp.int32),
        pltpu.SemaphoreType.DMA,
        pltpu.SemaphoreType.DMA,
        pltpu.SemaphoreType.DMA,
        pltpu.SemaphoreType.DMA,
        pltpu.SemaphoreType.DMA,
        pltpu.SemaphoreType.DMA,
        pltpu.SemaphoreType.DMA,
    ],
)


BR = 1000


def _norm_kernel(v_ref, sr_ref, tg_ref):
    v = v_ref[...]
    for gi_, oref in ((0, sr_ref), (1, tg_ref)):
        x = jnp.concatenate([v[gi_, 0], v[gi_, 1]], axis=1)
        nrm = jnp.sqrt(jnp.sum(x * x, axis=1, keepdims=True))
        oref[...] = x / jnp.maximum(nrm, 1e-12)


_norm = pl.pallas_call(
    _norm_kernel,
    grid=(N // BR,),
    in_specs=[pl.BlockSpec((2, NC, BR, DH), lambda i: (0, 0, i, 0))],
    out_specs=[pl.BlockSpec((BR, D), lambda i: (i, 0)),
               pl.BlockSpec((BR, D), lambda i: (i, 0))],
    out_shape=[jax.ShapeDtypeStruct((N, D), jnp.float32)] * 2,
)


def _pad_idx(idx, pad_base, pad_mod):
    blocks = idx.reshape(NS, EPT)
    toff = jnp.arange(NS, dtype=jnp.int32)[:, None] * 16
    pads = pad_base + (jnp.arange(EPAD, dtype=jnp.int32)[None, :] + toff) % pad_mod
    full = jnp.concatenate([blocks, pads.astype(idx.dtype)], axis=1)
    return full.reshape(NS * NCHUNK, ECHUNK)


def kernel(emb_sr, emb_tg, edge_index_sr, edge_index_tg):
    ssr = edge_index_sr[0].astype(jnp.int32)
    dsr = edge_index_sr[1].astype(jnp.int32)
    stg = edge_index_tg[0].astype(jnp.int32)
    dtg = edge_index_tg[1].astype(jnp.int32)
    sp0 = _pad_idx(ssr, N, NPAD - N)
    dp0 = _pad_idx(dsr, TRASH, ACC_ROWS - TRASH)
    sp1 = _pad_idx(stg, N, NPAD - N)
    dp1 = _pad_idx(dtg, TRASH, ACC_ROWS - TRASH)
    deg = _deg(dp0, dp1)
    v2, _ = _sc_gcn(emb_sr, emb_tg, sp0, dp0, sp1, dp1, deg)
    sr, tg = _norm(v2)
    return (sr, tg)


# speedup vs baseline: 14.2055x; 1.2229x over previous
import jax
import jax.numpy as jnp
from jax import lax
from jax.experimental import pallas as pl
from jax.experimental.pallas import tpu as pltpu
from jax.experimental.pallas import tpu_sc as plsc

N = 10000
D = 256
E = 160000
NC = 2
NS = 16
LANES = 16
DH = D // NC
NPAD = 10240
RPT = NPAD // NS
EPT = E // NS
ECHUNK = 128
NCHUNK = 80
EPAD = NCHUNK * ECHUNK - EPT
ACC_ROWS = 10496
ZPT = ACC_ROWS // NS
TRASH = NPAD
HR = NPAD // 128
HRP = 88


def _rsqrt16(x):
    i = plsc.bitcast(x, jnp.int32)
    i = jnp.int32(0x5F3759DF) - lax.shift_right_logical(i, 1)
    y = plsc.bitcast(i, jnp.float32)
    for _ in range(3):
        y = y * (1.5 - 0.5 * x * y * y)
    return y


def _deg_body(dp0, dp1, degout, hists_sp, hist, idxbuf, dbuf):
    c = lax.axis_index("c")
    t = lax.axis_index("s")
    rbase = t * RPT
    zero16 = jnp.zeros((LANES,), jnp.float32)
    one16 = jnp.full((LANES,), 1.0, jnp.float32)

    for g in range(2):
        dp = (dp0, dp1)[g]

        def _zh(i, _):
            hist[i // 8, pl.ds((i % 8) * LANES, LANES)] = zero16
            return 0
        lax.fori_loop(0, HRP * 8, _zh, 0)

        pltpu.sync_copy(dp.at[pl.ds(t * NCHUNK, NCHUNK)], idxbuf)

        def _hs(i, _):
            v = idxbuf[i // 8, pl.ds((i % 8) * LANES, LANES)]
            r = lax.shift_right_logical(v, 7)
            cc = lax.bitwise_and(v, 127)
            plsc.addupdate_scatter(hist, [r, cc], one16)
            return 0
        lax.fori_loop(0, NCHUNK * 8, _hs, 0)

        plsc.subcore_barrier()
        pltpu.sync_copy(hist.at[pl.ds(0, HR)], hists_sp.at[pl.ds(t * HR, HR)])
        plsc.subcore_barrier()
        for tt in range(NS):
            pltpu.sync_copy(hists_sp.at[pl.ds(tt * HR + t * 5, 5)],
                            hist.at[pl.ds(tt * 5, 5)])

        def _sum(kk, _):
            s = one16
            for tt in range(NS):
                s = s + hist[tt * 5 + kk // 8, pl.ds((kk % 8) * LANES, LANES)]
            dbuf[pl.ds(kk * LANES, LANES)] = s
            return 0
        lax.fori_loop(0, RPT // LANES, _sum, 0)
        @pl.when(c == 0)
        def _():
            pltpu.sync_copy(dbuf, degout.at[g].at[pl.ds(rbase, RPT)])
        plsc.subcore_barrier()


_deg = pl.kernel(
    _deg_body,
    out_type=jax.ShapeDtypeStruct((2, NPAD), jnp.float32),
    mesh=plsc.VectorSubcoreMesh(core_axis_name="c", subcore_axis_name="s",
                                num_cores=NC, num_subcores=NS),
    compiler_params=pltpu.CompilerParams(needs_layout_passes=False),
    scratch_types=[
        pltpu.VMEM_SHARED((NS * HR, 128), jnp.float32),
        pltpu.VMEM((HRP, 128), jnp.float32),
        pltpu.VMEM((NCHUNK, ECHUNK), jnp.int32),
        pltpu.VMEM((RPT,), jnp.float32),
    ],
)


def _sc_body(emb_sr, emb_tg, sp0, dp0, sp1, dp1, deg, v2, ubh,
             acc, dgbuf, rbuf, ga, gb, gi, si,
             is0, is1, is2, is3, gsem, ss0, ss1):
    c = lax.axis_index("c")
    t = lax.axis_index("s")
    rbase = t * RPT
    cbase = t * NCHUNK
    nblk = jnp.where(t == NS - 1, (N - (NS - 1) * RPT) // LANES, RPT // LANES)

    zero16 = jnp.zeros((LANES,), jnp.float32)
    isems = (is0, is1, is2, is3)
    gbufs = (ga, gb)
    ssems = (ss0, ss1)
    nblk64 = jnp.where(t == NS - 1, 6, RPT // 64)
    hbm_d64 = ubh.at[0].at[0].at[pl.ds(0, 64)]

    for g in range(2):
        emb = (emb_sr, emb_tg)[g]
        sp = (sp0, sp1)[g]
        dp = (dp0, dp1)[g]

        pltpu.sync_copy(deg.at[g].at[pl.ds(rbase, RPT)], dgbuf)

        def _rs(kk, _):
            rbuf[pl.ds(kk * LANES, LANES)] = _rsqrt16(
                dgbuf[pl.ds(kk * LANES, LANES)])
            return 0
        lax.fori_loop(0, RPT // LANES, _rs, 0)

        def _u0(b, _):
            r0 = rbase + b * 64
            din = pltpu.async_copy(
                emb.at[pl.ds(r0, 64), pl.ds(c * DH, DH)],
                gb.at[pl.ds(0, 64)], is0)

            @pl.when(b > 0)
            def _():
                pltpu.make_async_copy(hbm_d64, ga.at[pl.ds(0, 64)],
                                      ss0).wait()
            din.wait()

            def _row(i, _):
                s = plsc.load_gather(
                    rbuf, [jnp.full((LANES,), b * 64 + i, jnp.int32)])
                for k in range(DH // LANES):
                    ga[i, pl.ds(k * LANES, LANES)] = (
                        gb[i, pl.ds(k * LANES, LANES)] * s)
                return 0
            lax.fori_loop(0, 64, _row, 0)
            pltpu.async_copy(ga.at[pl.ds(0, 64)],
                             ubh.at[0].at[c].at[pl.ds(r0, 64)], ss0)
            return 0
        lax.fori_loop(0, nblk64, _u0, 0)
        pltpu.make_async_copy(hbm_d64, ga.at[pl.ds(0, 64)], ss0).wait()

        @pl.when(t == NS - 1)
        def _(emb=emb):
            r0 = rbase + 6 * 64
            pltpu.sync_copy(emb.at[pl.ds(r0, 16), pl.ds(c * DH, DH)],
                            gb.at[pl.ds(0, 16)])

            def _row(i, _):
                s = plsc.load_gather(
                    rbuf, [jnp.full((LANES,), 6 * 64 + i, jnp.int32)])
                for k in range(DH // LANES):
                    ga[i, pl.ds(k * LANES, LANES)] = (
                        gb[i, pl.ds(k * LANES, LANES)] * s)
                return 0
            lax.fori_loop(0, LANES, _row, 0)
            pltpu.sync_copy(ga.at[pl.ds(0, 16)],
                            ubh.at[0].at[c].at[pl.ds(r0, 16)])

        for l in range(2):
            plsc.subcore_barrier()

            def _zb(i, _):
                ga[i // 8, pl.ds((i % 8) * LANES, LANES)] = zero16
                return 0
            lax.fori_loop(0, 64 * 8, _zb, 0)

            def _za(k2, _):
                pltpu.sync_copy(ga.at[pl.ds(0, 64)],
                                acc.at[pl.ds(t * ZPT + k2 * 64, 64)])
                return 0
            lax.fori_loop(0, ZPT // 64, _za, 0)
            pltpu.sync_copy(ga.at[pl.ds(0, 16)],
                            acc.at[pl.ds(t * ZPT + (ZPT // 64) * 64, 16)])
            plsc.subcore_barrier()

            uin = ubh.at[l].at[c]
            hbm_dummy = uin.at[pl.ds(0, ECHUNK)]

            def issue_idx(j, q, sp=sp, dp=dp):
                pltpu.async_copy(sp.at[cbase + j], gi.at[q], isems[q])
                pltpu.async_copy(dp.at[cbase + j], si.at[q], isems[q])

            def do_chunk(j, slot, prefetch, wait_scatter,
                         uin=uin, hbm_dummy=hbm_dummy, sp=sp, dp=dp,
                         issue_idx=issue_idx):
                p = slot % 2
                q = slot % 4
                if wait_scatter:
                    pltpu.make_async_copy(hbm_dummy, gbufs[p],
                                          ssems[p]).wait()
                pltpu.make_async_copy(sp.at[0], gi.at[q], isems[q]).wait()
                pltpu.make_async_copy(dp.at[0], si.at[q], isems[q]).wait()
                gd = pltpu.async_copy(uin.at[gi.at[q]], gbufs[p], gsem)
                if prefetch is not None:
                    issue_idx(prefetch, (q + 2) % 4)
                gd.wait()
                pltpu.async_copy(gbufs[p], acc.at[si.at[q]], ssems[p],
                                 add=True)

            issue_idx(0, 0)
            issue_idx(1, 1)
            do_chunk(0, 0, 2, False)
            do_chunk(1, 1, 3, False)

            def _pipe(jo, _):
                jb = 2 + jo * 4
                for b in range(4):
                    do_chunk(jb + b, 2 + b, jb + b + 2, True)
                return 0
            lax.fori_loop(0, (NCHUNK - 4) // 4, _pipe, 0)

            do_chunk(NCHUNK - 2, 2, None, True)
            do_chunk(NCHUNK - 1, 3, None, True)
            pltpu.make_async_copy(hbm_dummy, ga, ss0).wait()
            pltpu.make_async_copy(hbm_dummy, gb, ss1).wait()
            plsc.subcore_barrier()

            out = ubh.at[1].at[c] if l == 0 else v2.at[g].at[c]

            def _ep(b, _, uin=uin, out=out, l=l):
                r0 = rbase + b * 64
                da = pltpu.async_copy(acc.at[pl.ds(r0, 64)],
                                      gb.at[pl.ds(0, 64)], is0)
                du = pltpu.async_copy(uin.at[pl.ds(r0, 64)],
                                      gb.at[pl.ds(64, 64)], is1)

                @pl.when(b > 0)
                def _():
                    pltpu.make_async_copy(hbm_d64, ga.at[pl.ds(0, 64)],
                                          ss0).wait()
                da.wait()
                du.wait()

                def _row(i, _):
                    if l == 0:
                        dd = plsc.load_gather(
                            dgbuf,
                            [jnp.full((LANES,), b * 64 + i, jnp.int32)])
                        s = 1.0 / dd
                        for k in range(DH // LANES):
                            ga[i, pl.ds(k * LANES, LANES)] = (
                                gb[i, pl.ds(k * LANES, LANES)]
                                + gb[64 + i, pl.ds(k * LANES, LANES)]) * s
                    else:
                        for k in range(DH // LANES):
                            ga[i, pl.ds(k * LANES, LANES)] = (
                                gb[i, pl.ds(k * LANES, LANES)]
                                + gb[64 + i, pl.ds(k * LANES, LANES)])
                    return 0
                lax.fori_loop(0, 64, _row, 0)
                pltpu.async_copy(ga.at[pl.ds(0, 64)],
                                 out.at[pl.ds(r0, 64)], ss0)
                return 0
            lax.fori_loop(0, nblk64, _ep, 0)
            pltpu.make_async_copy(hbm_d64, ga.at[pl.ds(0, 64)], ss0).wait()

            @pl.when(t == NS - 1)
            def _(uin=uin, out=out, l=l):
                r0 = rbase + 6 * 64
                pltpu.sync_copy(acc.at[pl.ds(r0, 16)], gb.at[pl.ds(0, 16)])
                pltpu.sync_copy(uin.at[pl.ds(r0, 16)], gb.at[pl.ds(64, 16)])

                def _row(i, _):
                    if l == 0:
                        dd = plsc.load_gather(
                            dgbuf,
                            [jnp.full((LANES,), 6 * 64 + i, jnp.int32)])
                        s = 1.0 / dd
                        for k in range(DH // LANES):
                            ga[i, pl.ds(k * LANES, LANES)] = (
                                gb[i, pl.ds(k * LANES, LANES)]
                                + gb[64 + i, pl.ds(k * LANES, LANES)]) * s
                    else:
                        for k in range(DH // LANES):
                            ga[i, pl.ds(k * LANES, LANES)] = (
                                gb[i, pl.ds(k * LANES, LANES)]
                                + gb[64 + i, pl.ds(k * LANES, LANES)])
                    return 0
                lax.fori_loop(0, LANES, _row, 0)
                pltpu.sync_copy(ga.at[pl.ds(0, 16)], out.at[pl.ds(r0, 16)])


_sc_gcn = pl.kernel(
    _sc_body,
    out_type=(
        jax.ShapeDtypeStruct((2, NC, NPAD, DH), jnp.float32),
        jax.ShapeDtypeStruct((2, NC, NPAD, DH), jnp.float32),
    ),
    mesh=plsc.VectorSubcoreMesh(core_axis_name="c", subcore_axis_name="s",
                                num_cores=NC, num_subcores=NS),
    compiler_params=pltpu.CompilerParams(needs_layout_passes=False),
    scratch_types=[
        pltpu.VMEM_SHARED((ACC_ROWS, DH), jnp.float32),
        pltpu.VMEM((RPT,), jnp.float32),
        pltpu.VMEM((RPT,), jnp.float32),
        pltpu.VMEM((ECHUNK, DH), jnp.float32),
        pltpu.VMEM((ECHUNK, DH), jnp.float32),
        pltpu.VMEM((4, ECHUNK), jnp.int32),
        pltpu.VMEM((4, ECHUNK), jnp.int32),
        pltpu.SemaphoreType.DMA,
        pltpu.SemaphoreType.DMA,
        pltpu.SemaphoreType.DMA,
        pltpu.SemaphoreType.DMA,
        pltpu.SemaphoreType.DMA,
        pltpu.SemaphoreType.DMA,
        pltpu.SemaphoreType.DMA,
    ],
)


BR = 1000


def _norm_kernel(v_ref, sr_ref, tg_ref):
    v = v_ref[...]
    for gi_, oref in ((0, sr_ref), (1, tg_ref)):
        x = jnp.concatenate([v[gi_, 0], v[gi_, 1]], axis=1)
        nrm = jnp.sqrt(jnp.sum(x * x, axis=1, keepdims=True))
        oref[...] = x / jnp.maximum(nrm, 1e-12)


_norm = pl.pallas_call(
    _norm_kernel,
    grid=(N // BR,),
    in_specs=[pl.BlockSpec((2, NC, BR, DH), lambda i: (0, 0, i, 0))],
    out_specs=[pl.BlockSpec((BR, D), lambda i: (i, 0)),
               pl.BlockSpec((BR, D), lambda i: (i, 0))],
    out_shape=[jax.ShapeDtypeStruct((N, D), jnp.float32)] * 2,
)


def _pad_idx(idx, pad_base, pad_mod):
    blocks = idx.reshape(NS, EPT)
    toff = jnp.arange(NS, dtype=jnp.int32)[:, None] * 16
    pads = pad_base + (jnp.arange(EPAD, dtype=jnp.int32)[None, :] + toff) % pad_mod
    full = jnp.concatenate([blocks, pads.astype(idx.dtype)], axis=1)
    return full.reshape(NS * NCHUNK, ECHUNK)


def kernel(emb_sr, emb_tg, edge_index_sr, edge_index_tg):
    ssr = edge_index_sr[0].astype(jnp.int32)
    dsr = edge_index_sr[1].astype(jnp.int32)
    stg = edge_index_tg[0].astype(jnp.int32)
    dtg = edge_index_tg[1].astype(jnp.int32)
    sp0 = _pad_idx(ssr, N, NPAD - N)
    dp0 = _pad_idx(dsr, TRASH, ACC_ROWS - TRASH)
    sp1 = _pad_idx(stg, N, NPAD - N)
    dp1 = _pad_idx(dtg, TRASH, ACC_ROWS - TRASH)
    deg = _deg(dp0, dp1)
    v2, _ = _sc_gcn(emb_sr, emb_tg, sp0, dp0, sp1, dp1, deg)
    sr, tg = _norm(v2)
    return (sr, tg)


# speedup vs baseline: 16.3637x; 1.1519x over previous
import jax
import jax.numpy as jnp
from jax import lax
from jax.experimental import pallas as pl
from jax.experimental.pallas import tpu as pltpu
from jax.experimental.pallas import tpu_sc as plsc

N = 10000
D = 256
E = 160000
NC = 2
NS = 16
LANES = 16
DH = D // NC
NPAD = 10240
RPT = NPAD // NS
EPT = E // NS
ECHUNK = 128
NCHUNK = 80
EPAD = NCHUNK * ECHUNK - EPT
ACC_ROWS = 10496
ZPT = ACC_ROWS // NS
TRASH = NPAD
HR = NPAD // 128
HRP = 88


def _rsqrt16(x):
    i = plsc.bitcast(x, jnp.int32)
    i = jnp.int32(0x5F3759DF) - lax.shift_right_logical(i, 1)
    y = plsc.bitcast(i, jnp.float32)
    for _ in range(3):
        y = y * (1.5 - 0.5 * x * y * y)
    return y


def _deg_body(dp0, dp1, degout, hists_sp, hist, idxbuf, dbuf):
    c = lax.axis_index("c")
    t = lax.axis_index("s")
    rbase = t * RPT
    zero16 = jnp.zeros((LANES,), jnp.float32)
    one16 = jnp.full((LANES,), 1.0, jnp.float32)

    for g in range(2):
        dp = (dp0, dp1)[g]

        def _zh(i, _):
            hist[i // 8, pl.ds((i % 8) * LANES, LANES)] = zero16
            return 0
        lax.fori_loop(0, HRP * 8, _zh, 0)

        pltpu.sync_copy(dp.at[pl.ds(t * NCHUNK, NCHUNK)], idxbuf)

        def _hs(i, _):
            v = idxbuf[i // 8, pl.ds((i % 8) * LANES, LANES)]
            r = lax.shift_right_logical(v, 7)
            cc = lax.bitwise_and(v, 127)
            plsc.addupdate_scatter(hist, [r, cc], one16)
            return 0
        lax.fori_loop(0, NCHUNK * 8, _hs, 0)

        plsc.subcore_barrier()
        pltpu.sync_copy(hist.at[pl.ds(0, HR)], hists_sp.at[pl.ds(t * HR, HR)])
        plsc.subcore_barrier()
        for tt in range(NS):
            pltpu.sync_copy(hists_sp.at[pl.ds(tt * HR + t * 5, 5)],
                            hist.at[pl.ds(tt * 5, 5)])

        def _sum(kk, _):
            s = one16
            for tt in range(NS):
                s = s + hist[tt * 5 + kk // 8, pl.ds((kk % 8) * LANES, LANES)]
            dbuf[pl.ds(kk * LANES, LANES)] = s
            return 0
        lax.fori_loop(0, RPT // LANES, _sum, 0)
        @pl.when(c == 0)
        def _():
            pltpu.sync_copy(dbuf, degout.at[g].at[pl.ds(rbase, RPT)])
        plsc.subcore_barrier()


_deg = pl.kernel(
    _deg_body,
    out_type=jax.ShapeDtypeStruct((2, NPAD), jnp.float32),
    mesh=plsc.VectorSubcoreMesh(core_axis_name="c", subcore_axis_name="s",
                                num_cores=NC, num_subcores=NS),
    compiler_params=pltpu.CompilerParams(needs_layout_passes=False),
    scratch_types=[
        pltpu.VMEM_SHARED((NS * HR, 128), jnp.float32),
        pltpu.VMEM((HRP, 128), jnp.float32),
        pltpu.VMEM((NCHUNK, ECHUNK), jnp.int32),
        pltpu.VMEM((RPT,), jnp.float32),
    ],
)


def _sc_body(emb_sr, emb_tg, sp0, dp0, sp1, dp1, deg, v2, ubh,
             acc, dgbuf, rbuf, ga, gb, gi, si,
             is0, is1, is2, is3, gs0, gs1, ss0, ss1):
    c = lax.axis_index("c")
    t = lax.axis_index("s")
    rbase = t * RPT
    cbase = t * NCHUNK
    nblk = jnp.where(t == NS - 1, (N - (NS - 1) * RPT) // LANES, RPT // LANES)

    zero16 = jnp.zeros((LANES,), jnp.float32)
    isems = (is0, is1, is2, is3)
    gbufs = (ga, gb)
    gsems = (gs0, gs1)
    ssems = (ss0, ss1)
    nblk64 = jnp.where(t == NS - 1, 6, RPT // 64)
    hbm_d64 = ubh.at[0].at[0].at[pl.ds(0, 64)]

    for g in range(2):
        emb = (emb_sr, emb_tg)[g]
        sp = (sp0, sp1)[g]
        dp = (dp0, dp1)[g]

        pltpu.sync_copy(deg.at[g].at[pl.ds(rbase, RPT)], dgbuf)

        def _rs(kk, _):
            rbuf[pl.ds(kk * LANES, LANES)] = _rsqrt16(
                dgbuf[pl.ds(kk * LANES, LANES)])
            return 0
        lax.fori_loop(0, RPT // LANES, _rs, 0)

        def _u0(b, _):
            r0 = rbase + b * 64
            din = pltpu.async_copy(
                emb.at[pl.ds(r0, 64), pl.ds(c * DH, DH)],
                gb.at[pl.ds(0, 64)], is0)

            @pl.when(b > 0)
            def _():
                pltpu.make_async_copy(hbm_d64, ga.at[pl.ds(0, 64)],
                                      ss0).wait()
            din.wait()

            def _row(i, _):
                s = plsc.load_gather(
                    rbuf, [jnp.full((LANES,), b * 64 + i, jnp.int32)])
                for k in range(DH // LANES):
                    ga[i, pl.ds(k * LANES, LANES)] = (
                        gb[i, pl.ds(k * LANES, LANES)] * s)
                return 0
            lax.fori_loop(0, 64, _row, 0)
            pltpu.async_copy(ga.at[pl.ds(0, 64)],
                             ubh.at[0].at[c].at[pl.ds(r0, 64)], ss0)
            return 0
        lax.fori_loop(0, nblk64, _u0, 0)
        pltpu.make_async_copy(hbm_d64, ga.at[pl.ds(0, 64)], ss0).wait()

        @pl.when(t == NS - 1)
        def _(emb=emb):
            r0 = rbase + 6 * 64
            pltpu.sync_copy(emb.at[pl.ds(r0, 16), pl.ds(c * DH, DH)],
                            gb.at[pl.ds(0, 16)])

            def _row(i, _):
                s = plsc.load_gather(
                    rbuf, [jnp.full((LANES,), 6 * 64 + i, jnp.int32)])
                for k in range(DH // LANES):
                    ga[i, pl.ds(k * LANES, LANES)] = (
                        gb[i, pl.ds(k * LANES, LANES)] * s)
                return 0
            lax.fori_loop(0, LANES, _row, 0)
            pltpu.sync_copy(ga.at[pl.ds(0, 16)],
                            ubh.at[0].at[c].at[pl.ds(r0, 16)])

        for l in range(2):
            plsc.subcore_barrier()

            def _zb(i, _):
                ga[i // 8, pl.ds((i % 8) * LANES, LANES)] = zero16
                return 0
            lax.fori_loop(0, 64 * 8, _zb, 0)

            def _za(k2, _):
                pltpu.sync_copy(ga.at[pl.ds(0, 64)],
                                acc.at[pl.ds(t * ZPT + k2 * 64, 64)])
                return 0
            lax.fori_loop(0, ZPT // 64, _za, 0)
            pltpu.sync_copy(ga.at[pl.ds(0, 16)],
                            acc.at[pl.ds(t * ZPT + (ZPT // 64) * 64, 16)])
            plsc.subcore_barrier()

            uin = ubh.at[l].at[c]
            hbm_dummy = uin.at[pl.ds(0, ECHUNK)]

            def issue_idx(j, q, sp=sp, dp=dp):
                pltpu.async_copy(sp.at[cbase + j], gi.at[q], isems[q])
                pltpu.async_copy(dp.at[cbase + j], si.at[q], isems[q])

            def do_chunk(j, slot, prefetch, drain_scatter, scatter_prev,
                         uin=uin, hbm_dummy=hbm_dummy, sp=sp, dp=dp,
                         issue_idx=issue_idx):
                p = slot % 2
                q = slot % 4
                if drain_scatter:
                    pltpu.make_async_copy(hbm_dummy, gbufs[p],
                                          ssems[p]).wait()
                if prefetch is not None:
                    issue_idx(prefetch, (q + 2) % 4)
                pltpu.make_async_copy(sp.at[0], gi.at[q], isems[q]).wait()
                pltpu.make_async_copy(dp.at[0], si.at[q], isems[q]).wait()
                pltpu.async_copy(uin.at[gi.at[q]], gbufs[p], gsems[p])
                if scatter_prev:
                    pltpu.make_async_copy(hbm_dummy, gbufs[1 - p],
                                          gsems[1 - p]).wait()
                    pltpu.async_copy(gbufs[1 - p], acc.at[si.at[(q + 3) % 4]],
                                     ssems[1 - p], add=True)

            issue_idx(0, 0)
            issue_idx(1, 1)
            do_chunk(0, 0, 2, False, False)
            do_chunk(1, 1, 3, False, True)

            def _pipe(jo, _):
                jb = 2 + jo * 4
                for b in range(4):
                    do_chunk(jb + b, 2 + b, jb + b + 2, True, True)
                return 0
            lax.fori_loop(0, (NCHUNK - 4) // 4, _pipe, 0)

            do_chunk(NCHUNK - 2, 2, None, True, True)
            do_chunk(NCHUNK - 1, 3, None, True, True)
            pltpu.make_async_copy(hbm_dummy, gb, gsems[1]).wait()
            pltpu.async_copy(gb, acc.at[si.at[3]], ssems[1], add=True)
            pltpu.make_async_copy(hbm_dummy, ga, ss0).wait()
            pltpu.make_async_copy(hbm_dummy, gb, ss1).wait()
            plsc.subcore_barrier()

            out = ubh.at[1].at[c] if l == 0 else v2.at[g].at[c]

            def _ep(b, _, uin=uin, out=out, l=l):
                r0 = rbase + b * 64
                da = pltpu.async_copy(acc.at[pl.ds(r0, 64)],
                                      gb.at[pl.ds(0, 64)], is0)
                du = pltpu.async_copy(uin.at[pl.ds(r0, 64)],
                                      gb.at[pl.ds(64, 64)], is1)

                @pl.when(b > 0)
                def _():
                    pltpu.make_async_copy(hbm_d64, ga.at[pl.ds(0, 64)],
                                          ss0).wait()
                da.wait()
                du.wait()

                def _row(i, _):
                    if l == 0:
                        dd = plsc.load_gather(
                            dgbuf,
                            [jnp.full((LANES,), b * 64 + i, jnp.int32)])
                        s = 1.0 / dd
                        for k in range(DH // LANES):
                            ga[i, pl.ds(k * LANES, LANES)] = (
                                gb[i, pl.ds(k * LANES, LANES)]
                                + gb[64 + i, pl.ds(k * LANES, LANES)]) * s
                    else:
                        for k in range(DH // LANES):
                            ga[i, pl.ds(k * LANES, LANES)] = (
                                gb[i, pl.ds(k * LANES, LANES)]
                                + gb[64 + i, pl.ds(k * LANES, LANES)])
                    return 0
                lax.fori_loop(0, 64, _row, 0)
                pltpu.async_copy(ga.at[pl.ds(0, 64)],
                                 out.at[pl.ds(r0, 64)], ss0)
                return 0
            lax.fori_loop(0, nblk64, _ep, 0)
            pltpu.make_async_copy(hbm_d64, ga.at[pl.ds(0, 64)], ss0).wait()

            @pl.when(t == NS - 1)
            def _(uin=uin, out=out, l=l):
                r0 = rbase + 6 * 64
                pltpu.sync_copy(acc.at[pl.ds(r0, 16)], gb.at[pl.ds(0, 16)])
                pltpu.sync_copy(uin.at[pl.ds(r0, 16)], gb.at[pl.ds(64, 16)])

                def _row(i, _):
                    if l == 0:
                        dd = plsc.load_gather(
                            dgbuf,
                            [jnp.full((LANES,), 6 * 64 + i, jnp.int32)])
                        s = 1.0 / dd
                        for k in range(DH // LANES):
                            ga[i, pl.ds(k * LANES, LANES)] = (
                                gb[i, pl.ds(k * LANES, LANES)]
                                + gb[64 + i, pl.ds(k * LANES, LANES)]) * s
                    else:
                        for k in range(DH // LANES):
                            ga[i, pl.ds(k * LANES, LANES)] = (
                                gb[i, pl.ds(k * LANES, LANES)]
                                + gb[64 + i, pl.ds(k * LANES, LANES)])
                    return 0
                lax.fori_loop(0, LANES, _row, 0)
                pltpu.sync_copy(ga.at[pl.ds(0, 16)], out.at[pl.ds(r0, 16)])


_sc_gcn = pl.kernel(
    _sc_body,
    out_type=(
        jax.ShapeDtypeStruct((2, NC, NPAD, DH), jnp.float32),
        jax.ShapeDtypeStruct((2, NC, NPAD, DH), jnp.float32),
    ),
    mesh=plsc.VectorSubcoreMesh(core_axis_name="c", subcore_axis_name="s",
                                num_cores=NC, num_subcores=NS),
    compiler_params=pltpu.CompilerParams(needs_layout_passes=False),
    scratch_types=[
        pltpu.VMEM_SHARED((ACC_ROWS, DH), jnp.float32),
        pltpu.VMEM((RPT,), jnp.float32),
        pltpu.VMEM((RPT,), jnp.float32),
        pltpu.VMEM((ECHUNK, DH), jnp.float32),
        pltpu.VMEM((ECHUNK, DH), jnp.float32),
        pltpu.VMEM((4, ECHUNK), jnp.int32),
        pltpu.VMEM((4, ECHUNK), jnp.int32),
        pltpu.SemaphoreType.DMA,
        pltpu.SemaphoreType.DMA,
        pltpu.SemaphoreType.DMA,
        pltpu.SemaphoreType.DMA,
        pltpu.SemaphoreType.DMA,
        pltpu.SemaphoreType.DMA,
        pltpu.SemaphoreType.DMA,
        pltpu.SemaphoreType.DMA,
    ],
)


BR = 1000


def _norm_kernel(v_ref, sr_ref, tg_ref):
    v = v_ref[...]
    for gi_, oref in ((0, sr_ref), (1, tg_ref)):
        x = jnp.concatenate([v[gi_, 0], v[gi_, 1]], axis=1)
        nrm = jnp.sqrt(jnp.sum(x * x, axis=1, keepdims=True))
        oref[...] = x / jnp.maximum(nrm, 1e-12)


_norm = pl.pallas_call(
    _norm_kernel,
    grid=(N // BR,),
    in_specs=[pl.BlockSpec((2, NC, BR, DH), lambda i: (0, 0, i, 0))],
    out_specs=[pl.BlockSpec((BR, D), lambda i: (i, 0)),
               pl.BlockSpec((BR, D), lambda i: (i, 0))],
    out_shape=[jax.ShapeDtypeStruct((N, D), jnp.float32)] * 2,
)


def _pad_idx(idx, pad_base, pad_mod):
    blocks = idx.reshape(NS, EPT)
    toff = jnp.arange(NS, dtype=jnp.int32)[:, None] * 16
    pads = pad_base + (jnp.arange(EPAD, dtype=jnp.int32)[None, :] + toff) % pad_mod
    full = jnp.concatenate([blocks, pads.astype(idx.dtype)], axis=1)
    return full.reshape(NS * NCHUNK, ECHUNK)


def kernel(emb_sr, emb_tg, edge_index_sr, edge_index_tg):
    ssr = edge_index_sr[0].astype(jnp.int32)
    dsr = edge_index_sr[1].astype(jnp.int32)
    stg = edge_index_tg[0].astype(jnp.int32)
    dtg = edge_index_tg[1].astype(jnp.int32)
    sp0 = _pad_idx(ssr, N, NPAD - N)
    dp0 = _pad_idx(dsr, TRASH, ACC_ROWS - TRASH)
    sp1 = _pad_idx(stg, N, NPAD - N)
    dp1 = _pad_idx(dtg, TRASH, ACC_ROWS - TRASH)
    deg = _deg(dp0, dp1)
    v2, _ = _sc_gcn(emb_sr, emb_tg, sp0, dp0, sp1, dp1, deg)
    sr, tg = _norm(v2)
    return (sr, tg)


# speedup vs baseline: 16.7525x; 1.0238x over previous
import jax
import jax.numpy as jnp
from jax import lax
from jax.experimental import pallas as pl
from jax.experimental.pallas import tpu as pltpu
from jax.experimental.pallas import tpu_sc as plsc

N = 10000
D = 256
E = 160000
NC = 2
NS = 16
LANES = 16
DH = D // NC
NPAD = 10240
RPT = NPAD // NS
EPT = E // NS
ECHUNK = 128
NCHUNK = 85
EPAD = 240
NCROW = 88
ACC_ROWS = 10496
ZPT = ACC_ROWS // NS
TRASH = NPAD
HR = NPAD // 128
HRP = 88


def _rsqrt16(x):
    i = plsc.bitcast(x, jnp.int32)
    i = jnp.int32(0x5F3759DF) - lax.shift_right_logical(i, 1)
    y = plsc.bitcast(i, jnp.float32)
    for _ in range(3):
        y = y * (1.5 - 0.5 * x * y * y)
    return y


def _deg_body(dp0, dp1, degout, hists_sp, hist, idxbuf, dbuf):
    c = lax.axis_index("c")
    t = lax.axis_index("s")
    rbase = t * RPT
    zero16 = jnp.zeros((LANES,), jnp.float32)
    one16 = jnp.full((LANES,), 1.0, jnp.float32)

    for g in range(2):
        dp = (dp0, dp1)[g]

        def _zh(i, _):
            hist[i // 8, pl.ds((i % 8) * LANES, LANES)] = zero16
            return 0
        lax.fori_loop(0, HRP * 8, _zh, 0)

        pltpu.sync_copy(dp.at[pl.ds(t * NCROW, NCROW)], idxbuf)

        def _hs(i, _):
            v = idxbuf[i // 8, pl.ds((i % 8) * LANES, LANES)]
            r = lax.shift_right_logical(v, 7)
            cc = lax.bitwise_and(v, 127)
            plsc.addupdate_scatter(hist, [r, cc], one16)
            return 0
        lax.fori_loop(0, NCHUNK * 8, _hs, 0)

        plsc.subcore_barrier()
        pltpu.sync_copy(hist.at[pl.ds(0, HR)], hists_sp.at[pl.ds(t * HR, HR)])
        plsc.subcore_barrier()
        for tt in range(NS):
            pltpu.sync_copy(hists_sp.at[pl.ds(tt * HR + t * 5, 5)],
                            hist.at[pl.ds(tt * 5, 5)])

        def _sum(kk, _):
            s = zero16
            for tt in range(NS):
                s = s + hist[tt * 5 + kk // 8, pl.ds((kk % 8) * LANES, LANES)]
            dbuf[pl.ds(kk * LANES, LANES)] = s
            return 0
        lax.fori_loop(0, RPT // LANES, _sum, 0)
        @pl.when(c == 0)
        def _():
            pltpu.sync_copy(dbuf, degout.at[g].at[pl.ds(rbase, RPT)])
        plsc.subcore_barrier()


_deg = pl.kernel(
    _deg_body,
    out_type=jax.ShapeDtypeStruct((2, NPAD), jnp.float32),
    mesh=plsc.VectorSubcoreMesh(core_axis_name="c", subcore_axis_name="s",
                                num_cores=NC, num_subcores=NS),
    compiler_params=pltpu.CompilerParams(needs_layout_passes=False),
    scratch_types=[
        pltpu.VMEM_SHARED((NS * HR, 128), jnp.float32),
        pltpu.VMEM((HRP, 128), jnp.float32),
        pltpu.VMEM((NCROW, ECHUNK), jnp.int32),
        pltpu.VMEM((RPT,), jnp.float32),
    ],
)


def _sc_body(emb_sr, emb_tg, sp0, dp0, sp1, dp1, deg, v2, ubh,
             acc, dgbuf, rbuf, ga, gb, gi, si,
             is0, is1, is2, is3, gs0, gs1, ss0, ss1):
    c = lax.axis_index("c")
    t = lax.axis_index("s")
    rbase = t * RPT
    cbase = t * NCROW
    nblk = jnp.where(t == NS - 1, (N - (NS - 1) * RPT) // LANES, RPT // LANES)

    zero16 = jnp.zeros((LANES,), jnp.float32)
    isems = (is0, is1, is2, is3)
    gbufs = (ga, gb)
    gsems = (gs0, gs1)
    ssems = (ss0, ss1)
    nblk64 = jnp.where(t == NS - 1, 6, RPT // 64)
    hbm_d64 = ubh.at[0].at[0].at[pl.ds(0, 64)]

    for g in range(2):
        emb = (emb_sr, emb_tg)[g]
        sp = (sp0, sp1)[g]
        dp = (dp0, dp1)[g]

        pltpu.sync_copy(deg.at[g].at[pl.ds(rbase, RPT)], dgbuf)

        def _rs(kk, _):
            rbuf[pl.ds(kk * LANES, LANES)] = _rsqrt16(
                dgbuf[pl.ds(kk * LANES, LANES)])
            return 0
        lax.fori_loop(0, RPT // LANES, _rs, 0)

        def _u0(b, _):
            r0 = rbase + b * 64
            din = pltpu.async_copy(
                emb.at[pl.ds(r0, 64), pl.ds(c * DH, DH)],
                gb.at[pl.ds(0, 64)], is0)

            @pl.when(b > 0)
            def _():
                pltpu.make_async_copy(hbm_d64, ga.at[pl.ds(0, 64)],
                                      ss0).wait()
            din.wait()

            def _row(i, _):
                s = plsc.load_gather(
                    rbuf, [jnp.full((LANES,), b * 64 + i, jnp.int32)])
                for k in range(DH // LANES):
                    ga[i, pl.ds(k * LANES, LANES)] = (
                        gb[i, pl.ds(k * LANES, LANES)] * s)
                return 0
            lax.fori_loop(0, 64, _row, 0)
            pltpu.async_copy(ga.at[pl.ds(0, 64)],
                             ubh.at[0].at[c].at[pl.ds(r0, 64)], ss0)
            return 0
        lax.fori_loop(0, nblk64, _u0, 0)
        pltpu.make_async_copy(hbm_d64, ga.at[pl.ds(0, 64)], ss0).wait()

        @pl.when(t == NS - 1)
        def _(emb=emb):
            r0 = rbase + 6 * 64
            pltpu.sync_copy(emb.at[pl.ds(r0, 16), pl.ds(c * DH, DH)],
                            gb.at[pl.ds(0, 16)])

            def _row(i, _):
                s = plsc.load_gather(
                    rbuf, [jnp.full((LANES,), 6 * 64 + i, jnp.int32)])
                for k in range(DH // LANES):
                    ga[i, pl.ds(k * LANES, LANES)] = (
                        gb[i, pl.ds(k * LANES, LANES)] * s)
                return 0
            lax.fori_loop(0, LANES, _row, 0)
            pltpu.sync_copy(ga.at[pl.ds(0, 16)],
                            ubh.at[0].at[c].at[pl.ds(r0, 16)])

        for l in range(2):
            plsc.subcore_barrier()

            def _zb(i, _):
                ga[i // 8, pl.ds((i % 8) * LANES, LANES)] = zero16
                return 0
            lax.fori_loop(0, 64 * 8, _zb, 0)

            def _za(k2, _):
                pltpu.sync_copy(ga.at[pl.ds(0, 64)],
                                acc.at[pl.ds(t * ZPT + k2 * 64, 64)])
                return 0
            lax.fori_loop(0, ZPT // 64, _za, 0)
            pltpu.sync_copy(ga.at[pl.ds(0, 16)],
                            acc.at[pl.ds(t * ZPT + (ZPT // 64) * 64, 16)])
            plsc.subcore_barrier()

            uin = ubh.at[l].at[c]
            hbm_dummy = uin.at[pl.ds(0, ECHUNK)]

            def issue_idx(j, q, sp=sp, dp=dp):
                pltpu.async_copy(sp.at[cbase + j], gi.at[q], isems[q])
                pltpu.async_copy(dp.at[cbase + j], si.at[q], isems[q])

            def do_chunk(j, slot, prefetch, drain_scatter, scatter_prev,
                         uin=uin, hbm_dummy=hbm_dummy, sp=sp, dp=dp,
                         issue_idx=issue_idx):
                p = slot % 2
                q = slot % 4
                if drain_scatter:
                    pltpu.make_async_copy(hbm_dummy, gbufs[p],
                                          ssems[p]).wait()
                if prefetch is not None:
                    issue_idx(prefetch, (q + 2) % 4)
                pltpu.make_async_copy(sp.at[0], gi.at[q], isems[q]).wait()
                pltpu.make_async_copy(dp.at[0], si.at[q], isems[q]).wait()
                pltpu.async_copy(uin.at[gi.at[q]], gbufs[p], gsems[p])
                if scatter_prev:
                    pltpu.make_async_copy(hbm_dummy, gbufs[1 - p],
                                          gsems[1 - p]).wait()
                    pltpu.async_copy(gbufs[1 - p], acc.at[si.at[(q + 3) % 4]],
                                     ssems[1 - p], add=True)

            issue_idx(0, 0)
            issue_idx(1, 1)
            do_chunk(0, 0, 2, False, False)
            do_chunk(1, 1, 3, False, True)

            def _pipe(jo, _):
                jb = 2 + jo * 4
                for b in range(4):
                    do_chunk(jb + b, 2 + b, jb + b + 2, True, True)
                return 0
            lax.fori_loop(0, 20, _pipe, 0)

            do_chunk(82, 82, 84, True, True)
            do_chunk(83, 83, None, True, True)
            do_chunk(84, 84, None, True, True)
            pltpu.make_async_copy(hbm_dummy, ga, gsems[0]).wait()
            pltpu.async_copy(ga, acc.at[si.at[0]], ssems[0], add=True)
            pltpu.make_async_copy(hbm_dummy, gb, ss1).wait()
            pltpu.make_async_copy(hbm_dummy, ga, ss0).wait()
            plsc.subcore_barrier()

            out = ubh.at[1].at[c] if l == 0 else v2.at[g].at[c]

            def _ep(b, _, out=out, l=l):
                r0 = rbase + b * 64
                da = pltpu.async_copy(acc.at[pl.ds(r0, 64)],
                                      gb.at[pl.ds(0, 64)], is0)

                @pl.when(b > 0)
                def _():
                    pltpu.make_async_copy(hbm_d64, ga.at[pl.ds(0, 64)],
                                          ss0).wait()
                da.wait()

                def _row(i, _):
                    if l == 0:
                        dd = plsc.load_gather(
                            dgbuf,
                            [jnp.full((LANES,), b * 64 + i, jnp.int32)])
                        sc = 1.0 / dd
                        for k in range(DH // LANES):
                            ga[i, pl.ds(k * LANES, LANES)] = (
                                gb[i, pl.ds(k * LANES, LANES)] * sc)
                    else:
                        for k in range(DH // LANES):
                            ga[i, pl.ds(k * LANES, LANES)] = (
                                gb[i, pl.ds(k * LANES, LANES)] + 0.0)
                    return 0
                lax.fori_loop(0, 64, _row, 0)
                pltpu.async_copy(ga.at[pl.ds(0, 64)],
                                 out.at[pl.ds(r0, 64)], ss0)
                return 0
            lax.fori_loop(0, nblk64, _ep, 0)
            pltpu.make_async_copy(hbm_d64, ga.at[pl.ds(0, 64)], ss0).wait()

            @pl.when(t == NS - 1)
            def _(out=out, l=l):
                r0 = rbase + 6 * 64
                pltpu.sync_copy(acc.at[pl.ds(r0, 16)], gb.at[pl.ds(0, 16)])

                def _row(i, _):
                    if l == 0:
                        dd = plsc.load_gather(
                            dgbuf,
                            [jnp.full((LANES,), 6 * 64 + i, jnp.int32)])
                        sc = 1.0 / dd
                        for k in range(DH // LANES):
                            ga[i, pl.ds(k * LANES, LANES)] = (
                                gb[i, pl.ds(k * LANES, LANES)] * sc)
                    else:
                        for k in range(DH // LANES):
                            ga[i, pl.ds(k * LANES, LANES)] = (
                                gb[i, pl.ds(k * LANES, LANES)] + 0.0)
                    return 0
                lax.fori_loop(0, LANES, _row, 0)
                pltpu.sync_copy(ga.at[pl.ds(0, 16)], out.at[pl.ds(r0, 16)])


_sc_gcn = pl.kernel(
    _sc_body,
    out_type=(
        jax.ShapeDtypeStruct((2, NC, NPAD, DH), jnp.float32),
        jax.ShapeDtypeStruct((2, NC, NPAD, DH), jnp.float32),
    ),
    mesh=plsc.VectorSubcoreMesh(core_axis_name="c", subcore_axis_name="s",
                                num_cores=NC, num_subcores=NS),
    compiler_params=pltpu.CompilerParams(needs_layout_passes=False),
    scratch_types=[
        pltpu.VMEM_SHARED((ACC_ROWS, DH), jnp.float32),
        pltpu.VMEM((RPT,), jnp.float32),
        pltpu.VMEM((RPT,), jnp.float32),
        pltpu.VMEM((ECHUNK, DH), jnp.float32),
        pltpu.VMEM((ECHUNK, DH), jnp.float32),
        pltpu.VMEM((4, ECHUNK), jnp.int32),
        pltpu.VMEM((4, ECHUNK), jnp.int32),
        pltpu.SemaphoreType.DMA,
        pltpu.SemaphoreType.DMA,
        pltpu.SemaphoreType.DMA,
        pltpu.SemaphoreType.DMA,
        pltpu.SemaphoreType.DMA,
        pltpu.SemaphoreType.DMA,
        pltpu.SemaphoreType.DMA,
        pltpu.SemaphoreType.DMA,
    ],
)


BR = 1000


def _norm_kernel(v_ref, sr_ref, tg_ref):
    v = v_ref[...]
    for gi_, oref in ((0, sr_ref), (1, tg_ref)):
        x = jnp.concatenate([v[gi_, 0], v[gi_, 1]], axis=1)
        nrm = jnp.sqrt(jnp.sum(x * x, axis=1, keepdims=True))
        oref[...] = x / jnp.maximum(nrm, 1e-12)


_norm = pl.pallas_call(
    _norm_kernel,
    grid=(N // BR,),
    in_specs=[pl.BlockSpec((2, NC, BR, DH), lambda i: (0, 0, i, 0))],
    out_specs=[pl.BlockSpec((BR, D), lambda i: (i, 0)),
               pl.BlockSpec((BR, D), lambda i: (i, 0))],
    out_shape=[jax.ShapeDtypeStruct((N, D), jnp.float32)] * 2,
)


def _pad_idx(idx, pad_base, pad_mod):
    blocks = idx.reshape(NS, EPT)
    toff = jnp.arange(NS, dtype=jnp.int32)[:, None] * 16
    pads = pad_base + (jnp.arange(EPAD, dtype=jnp.int32)[None, :] + toff) % pad_mod
    ident = (jnp.arange(NS, dtype=jnp.int32)[:, None] * RPT
             + jnp.arange(RPT, dtype=jnp.int32)[None, :])
    dummy = jnp.zeros((NS, (NCROW - NCHUNK) * ECHUNK), idx.dtype)
    full = jnp.concatenate([blocks, pads.astype(idx.dtype),
                            ident.astype(idx.dtype), dummy], axis=1)
    return full.reshape(NS * NCROW, ECHUNK)


def kernel(emb_sr, emb_tg, edge_index_sr, edge_index_tg):
    ssr = edge_index_sr[0].astype(jnp.int32)
    dsr = edge_index_sr[1].astype(jnp.int32)
    stg = edge_index_tg[0].astype(jnp.int32)
    dtg = edge_index_tg[1].astype(jnp.int32)
    sp0 = _pad_idx(ssr, N, NPAD - N)
    dp0 = _pad_idx(dsr, TRASH, ACC_ROWS - TRASH)
    sp1 = _pad_idx(stg, N, NPAD - N)
    dp1 = _pad_idx(dtg, TRASH, ACC_ROWS - TRASH)
    deg = _deg(dp0, dp1)
    v2, _ = _sc_gcn(emb_sr, emb_tg, sp0, dp0, sp1, dp1, deg)
    sr, tg = _norm(v2)
    return (sr, tg)


# speedup vs baseline: 16.9425x; 1.0113x over previous
import jax
import jax.numpy as jnp
from jax import lax
from jax.experimental import pallas as pl
from jax.experimental.pallas import tpu as pltpu
from jax.experimental.pallas import tpu_sc as plsc

N = 10000
D = 256
E = 160000
NC = 2
NS = 16
LANES = 16
DH = D // NC
NPAD = 10240
RPT = NPAD // NS
EPT = E // NS
ECHUNK = 128
NCHUNK = 85
EPAD = 240
NCROW = 88
ACC_ROWS = 10496
ZPT = ACC_ROWS // NS
TRASH = NPAD
HR = NPAD // 128
HRP = 88


def _rsqrt16(x):
    i = plsc.bitcast(x, jnp.int32)
    i = jnp.int32(0x5F3759DF) - lax.shift_right_logical(i, 1)
    y = plsc.bitcast(i, jnp.float32)
    for _ in range(3):
        y = y * (1.5 - 0.5 * x * y * y)
    return y


def _deg_body(dp0, dp1, degout, hists_sp, hist, idxbuf, dbuf):
    c = lax.axis_index("c")
    t = lax.axis_index("s")
    rbase = t * RPT
    zero16 = jnp.zeros((LANES,), jnp.float32)
    one16 = jnp.full((LANES,), 1.0, jnp.float32)

    for g in range(2):
        dp = (dp0, dp1)[g]

        def _zh(i, _):
            hist[i // 8, pl.ds((i % 8) * LANES, LANES)] = zero16
            return 0
        lax.fori_loop(0, HRP * 8, _zh, 0)

        pltpu.sync_copy(dp.at[pl.ds(t * NCROW, NCROW)], idxbuf)

        def _hs(i, _):
            v = idxbuf[i // 8, pl.ds((i % 8) * LANES, LANES)]
            r = lax.shift_right_logical(v, 7)
            cc = lax.bitwise_and(v, 127)
            plsc.addupdate_scatter(hist, [r, cc], one16)
            return 0
        lax.fori_loop(0, NCHUNK * 8, _hs, 0)

        plsc.subcore_barrier()
        pltpu.sync_copy(hist.at[pl.ds(0, HR)], hists_sp.at[pl.ds(t * HR, HR)])
        plsc.subcore_barrier()
        for tt in range(NS):
            pltpu.sync_copy(hists_sp.at[pl.ds(tt * HR + t * 5, 5)],
                            hist.at[pl.ds(tt * 5, 5)])

        def _sum(kk, _):
            s = zero16
            for tt in range(NS):
                s = s + hist[tt * 5 + kk // 8, pl.ds((kk % 8) * LANES, LANES)]
            dbuf[pl.ds(kk * LANES, LANES)] = s
            return 0
        lax.fori_loop(0, RPT // LANES, _sum, 0)
        @pl.when(c == 0)
        def _():
            pltpu.sync_copy(dbuf, degout.at[g].at[pl.ds(rbase, RPT)])
        plsc.subcore_barrier()


_deg = pl.kernel(
    _deg_body,
    out_type=jax.ShapeDtypeStruct((2, NPAD), jnp.float32),
    mesh=plsc.VectorSubcoreMesh(core_axis_name="c", subcore_axis_name="s",
                                num_cores=NC, num_subcores=NS),
    compiler_params=pltpu.CompilerParams(needs_layout_passes=False),
    scratch_types=[
        pltpu.VMEM_SHARED((NS * HR, 128), jnp.float32),
        pltpu.VMEM((HRP, 128), jnp.float32),
        pltpu.VMEM((NCROW, ECHUNK), jnp.int32),
        pltpu.VMEM((RPT,), jnp.float32),
    ],
)


def _sc_body(emb_sr, emb_tg, sp0, dp0, sp1, dp1, deg, v2, ubh,
             acc, dgbuf, rbuf, ga, gb, gi, si,
             is0, is1, is2, is3, gs0, gs1, ss0, ss1):
    c = lax.axis_index("c")
    t = lax.axis_index("s")
    rbase = t * RPT
    cbase = t * NCROW
    nblk = jnp.where(t == NS - 1, (N - (NS - 1) * RPT) // LANES, RPT // LANES)

    zero16 = jnp.zeros((LANES,), jnp.float32)
    isems = (is0, is1, is2, is3)
    gbufs = (ga, gb)
    gsems = (gs0, gs1)
    ssems = (ss0, ss1)
    nblk64 = jnp.where(t == NS - 1, 6, RPT // 64)
    hbm_d64 = ubh.at[0].at[0].at[pl.ds(0, 64)]

    for g in range(2):
        emb = (emb_sr, emb_tg)[g]
        sp = (sp0, sp1)[g]
        dp = (dp0, dp1)[g]

        pltpu.sync_copy(deg.at[g].at[pl.ds(rbase, RPT)], dgbuf)

        def _rs(kk, _):
            rbuf[pl.ds(kk * LANES, LANES)] = _rsqrt16(
                dgbuf[pl.ds(kk * LANES, LANES)])
            return 0
        lax.fori_loop(0, RPT // LANES, _rs, 0)

        def _u0(b, _):
            r0 = rbase + b * 64
            din = pltpu.async_copy(
                emb.at[pl.ds(r0, 64), pl.ds(c * DH, DH)],
                gb.at[pl.ds(0, 64)], is0)

            @pl.when(b > 0)
            def _():
                pltpu.make_async_copy(hbm_d64, ga.at[pl.ds(0, 64)],
                                      ss0).wait()
            din.wait()

            def _row(i, _):
                s = plsc.load_gather(
                    rbuf, [jnp.full((LANES,), b * 64 + i, jnp.int32)])
                for k in range(DH // LANES):
                    ga[i, pl.ds(k * LANES, LANES)] = (
                        gb[i, pl.ds(k * LANES, LANES)] * s)
                return 0
            lax.fori_loop(0, 64, _row, 0)
            pltpu.async_copy(ga.at[pl.ds(0, 64)],
                             ubh.at[0].at[c].at[pl.ds(r0, 64)], ss0)
            return 0
        lax.fori_loop(0, nblk64, _u0, 0)
        pltpu.make_async_copy(hbm_d64, ga.at[pl.ds(0, 64)], ss0).wait()

        @pl.when(t == NS - 1)
        def _(emb=emb):
            r0 = rbase + 6 * 64
            pltpu.sync_copy(emb.at[pl.ds(r0, 16), pl.ds(c * DH, DH)],
                            gb.at[pl.ds(0, 16)])

            def _row(i, _):
                s = plsc.load_gather(
                    rbuf, [jnp.full((LANES,), 6 * 64 + i, jnp.int32)])
                for k in range(DH // LANES):
                    ga[i, pl.ds(k * LANES, LANES)] = (
                        gb[i, pl.ds(k * LANES, LANES)] * s)
                return 0
            lax.fori_loop(0, LANES, _row, 0)
            pltpu.sync_copy(ga.at[pl.ds(0, 16)],
                            ubh.at[0].at[c].at[pl.ds(r0, 16)])

        for l in range(2):
            plsc.subcore_barrier()

            def _zb(i, _):
                ga[i // 8, pl.ds((i % 8) * LANES, LANES)] = zero16
                return 0
            lax.fori_loop(0, 64 * 8, _zb, 0)

            def _za(k2, _):
                pltpu.sync_copy(ga.at[pl.ds(0, 64)],
                                acc.at[pl.ds(t * ZPT + k2 * 64, 64)])
                return 0
            lax.fori_loop(0, ZPT // 64, _za, 0)
            pltpu.sync_copy(ga.at[pl.ds(0, 16)],
                            acc.at[pl.ds(t * ZPT + (ZPT // 64) * 64, 16)])
            plsc.subcore_barrier()

            uin = ubh.at[l].at[c]
            hbm_dummy = uin.at[pl.ds(0, ECHUNK)]

            def issue_idx(j, q, sp=sp, dp=dp):
                pltpu.async_copy(sp.at[cbase + j], gi.at[q], isems[q])
                pltpu.async_copy(dp.at[cbase + j], si.at[q], isems[q])

            def do_chunk(j, slot, prefetch, drain_scatter, scatter_prev,
                         uin=uin, hbm_dummy=hbm_dummy, sp=sp, dp=dp,
                         issue_idx=issue_idx):
                p = slot % 2
                q = slot % 4
                if drain_scatter:
                    pltpu.make_async_copy(hbm_dummy, gbufs[p],
                                          ssems[p]).wait()
                if prefetch is not None:
                    issue_idx(prefetch, (q + 2) % 4)
                pltpu.make_async_copy(sp.at[0], gi.at[q], isems[q]).wait()
                pltpu.make_async_copy(dp.at[0], si.at[q], isems[q]).wait()
                pltpu.async_copy(uin.at[gi.at[q]], gbufs[p], gsems[p])
                if scatter_prev:
                    pltpu.make_async_copy(hbm_dummy, gbufs[1 - p],
                                          gsems[1 - p]).wait()
                    pltpu.async_copy(gbufs[1 - p], acc.at[si.at[(q + 3) % 4]],
                                     ssems[1 - p], add=True)

            issue_idx(0, 0)
            issue_idx(1, 1)
            do_chunk(0, 0, 2, False, False)
            do_chunk(1, 1, 3, False, True)

            def _pipe(jo, _):
                jb = 2 + jo * 4
                for b in range(4):
                    do_chunk(jb + b, 2 + b, jb + b + 2, True, True)
                return 0
            lax.fori_loop(0, 20, _pipe, 0)

            do_chunk(82, 82, 84, True, True)
            do_chunk(83, 83, None, True, True)
            do_chunk(84, 84, None, True, True)
            pltpu.make_async_copy(hbm_dummy, ga, gsems[0]).wait()
            pltpu.async_copy(ga, acc.at[si.at[0]], ssems[0], add=True)
            pltpu.make_async_copy(hbm_dummy, gb, ss1).wait()
            pltpu.make_async_copy(hbm_dummy, ga, ss0).wait()
            plsc.subcore_barrier()

            out = ubh.at[1].at[c] if l == 0 else v2.at[g].at[c]

            if l == 0:
                def _ep(b, _, out=out):
                    r0 = rbase + b * 64
                    da = pltpu.async_copy(acc.at[pl.ds(r0, 64)],
                                          gb.at[pl.ds(0, 64)], is0)

                    @pl.when(b > 0)
                    def _():
                        pltpu.make_async_copy(hbm_d64, ga.at[pl.ds(0, 64)],
                                              ss0).wait()
                    da.wait()

                    def _row(i, _):
                        dd = plsc.load_gather(
                            dgbuf,
                            [jnp.full((LANES,), b * 64 + i, jnp.int32)])
                        sc = 1.0 / dd
                        for k in range(DH // LANES):
                            ga[i, pl.ds(k * LANES, LANES)] = (
                                gb[i, pl.ds(k * LANES, LANES)] * sc)
                        return 0
                    lax.fori_loop(0, 64, _row, 0)
                    pltpu.async_copy(ga.at[pl.ds(0, 64)],
                                     out.at[pl.ds(r0, 64)], ss0)
                    return 0
                lax.fori_loop(0, nblk64, _ep, 0)
                pltpu.make_async_copy(hbm_d64, ga.at[pl.ds(0, 64)],
                                      ss0).wait()

                @pl.when(t == NS - 1)
                def _(out=out):
                    r0 = rbase + 6 * 64
                    pltpu.sync_copy(acc.at[pl.ds(r0, 16)],
                                    gb.at[pl.ds(0, 16)])

                    def _row(i, _):
                        dd = plsc.load_gather(
                            dgbuf,
                            [jnp.full((LANES,), 6 * 64 + i, jnp.int32)])
                        sc = 1.0 / dd
                        for k in range(DH // LANES):
                            ga[i, pl.ds(k * LANES, LANES)] = (
                                gb[i, pl.ds(k * LANES, LANES)] * sc)
                        return 0
                    lax.fori_loop(0, LANES, _row, 0)
                    pltpu.sync_copy(ga.at[pl.ds(0, 16)],
                                    out.at[pl.ds(r0, 16)])
            else:
                def _cp2(bo, _, out=out):
                    for h, buf, osem, insem in ((0, ga, ss0, is0),
                                                (1, gb, ss1, is1)):
                        b = bo * 2 + h
                        r0 = rbase + b * 64

                        @pl.when(bo > 0)
                        def _(buf=buf, osem=osem):
                            pltpu.make_async_copy(hbm_d64,
                                                  buf.at[pl.ds(0, 64)],
                                                  osem).wait()
                        din = pltpu.async_copy(acc.at[pl.ds(r0, 64)],
                                               buf.at[pl.ds(0, 64)], insem)
                        din.wait()
                        pltpu.async_copy(buf.at[pl.ds(0, 64)],
                                         out.at[pl.ds(r0, 64)], osem)
                    return 0
                lax.fori_loop(0, nblk64 // 2, _cp2, 0)
                pltpu.make_async_copy(hbm_d64, ga.at[pl.ds(0, 64)],
                                      ss0).wait()
                pltpu.make_async_copy(hbm_d64, gb.at[pl.ds(0, 64)],
                                      ss1).wait()

                @pl.when(t == NS - 1)
                def _(out=out):
                    r0 = rbase + 6 * 64
                    pltpu.sync_copy(acc.at[pl.ds(r0, 16)],
                                    gb.at[pl.ds(0, 16)])
                    pltpu.sync_copy(gb.at[pl.ds(0, 16)],
                                    out.at[pl.ds(r0, 16)])


_sc_gcn = pl.kernel(
    _sc_body,
    out_type=(
        jax.ShapeDtypeStruct((2, NC, NPAD, DH), jnp.float32),
        jax.ShapeDtypeStruct((2, NC, NPAD, DH), jnp.float32),
    ),
    mesh=plsc.VectorSubcoreMesh(core_axis_name="c", subcore_axis_name="s",
                                num_cores=NC, num_subcores=NS),
    compiler_params=pltpu.CompilerParams(needs_layout_passes=False),
    scratch_types=[
        pltpu.VMEM_SHARED((ACC_ROWS, DH), jnp.float32),
        pltpu.VMEM((RPT,), jnp.float32),
        pltpu.VMEM((RPT,), jnp.float32),
        pltpu.VMEM((ECHUNK, DH), jnp.float32),
        pltpu.VMEM((ECHUNK, DH), jnp.float32),
        pltpu.VMEM((4, ECHUNK), jnp.int32),
        pltpu.VMEM((4, ECHUNK), jnp.int32),
        pltpu.SemaphoreType.DMA,
        pltpu.SemaphoreType.DMA,
        pltpu.SemaphoreType.DMA,
        pltpu.SemaphoreType.DMA,
        pltpu.SemaphoreType.DMA,
        pltpu.SemaphoreType.DMA,
        pltpu.SemaphoreType.DMA,
        pltpu.SemaphoreType.DMA,
    ],
)


BR = 1000


def _norm_kernel(v_ref, sr_ref, tg_ref):
    v = v_ref[...]
    for gi_, oref in ((0, sr_ref), (1, tg_ref)):
        x = jnp.concatenate([v[gi_, 0], v[gi_, 1]], axis=1)
        nrm = jnp.sqrt(jnp.sum(x * x, axis=1, keepdims=True))
        oref[...] = x / jnp.maximum(nrm, 1e-12)


_norm = pl.pallas_call(
    _norm_kernel,
    grid=(N // BR,),
    in_specs=[pl.BlockSpec((2, NC, BR, DH), lambda i: (0, 0, i, 0))],
    out_specs=[pl.BlockSpec((BR, D), lambda i: (i, 0)),
               pl.BlockSpec((BR, D), lambda i: (i, 0))],
    out_shape=[jax.ShapeDtypeStruct((N, D), jnp.float32)] * 2,
)


def _pad_idx(idx, pad_base, pad_mod):
    blocks = idx.reshape(NS, EPT)
    toff = jnp.arange(NS, dtype=jnp.int32)[:, None] * 16
    pads = pad_base + (jnp.arange(EPAD, dtype=jnp.int32)[None, :] + toff) % pad_mod
    ident = (jnp.arange(NS, dtype=jnp.int32)[:, None] * RPT
             + jnp.arange(RPT, dtype=jnp.int32)[None, :])
    dummy = jnp.zeros((NS, (NCROW - NCHUNK) * ECHUNK), idx.dtype)
    full = jnp.concatenate([blocks, pads.astype(idx.dtype),
                            ident.astype(idx.dtype), dummy], axis=1)
    return full.reshape(NS * NCROW, ECHUNK)


def kernel(emb_sr, emb_tg, edge_index_sr, edge_index_tg):
    ssr = edge_index_sr[0].astype(jnp.int32)
    dsr = edge_index_sr[1].astype(jnp.int32)
    stg = edge_index_tg[0].astype(jnp.int32)
    dtg = edge_index_tg[1].astype(jnp.int32)
    sp0 = _pad_idx(ssr, N, NPAD - N)
    dp0 = _pad_idx(dsr, TRASH, ACC_ROWS - TRASH)
    sp1 = _pad_idx(stg, N, NPAD - N)
    dp1 = _pad_idx(dtg, TRASH, ACC_ROWS - TRASH)
    deg = _deg(dp0, dp1)
    v2, _ = _sc_gcn(emb_sr, emb_tg, sp0, dp0, sp1, dp1, deg)
    sr, tg = _norm(v2)
    return (sr, tg)
